# baseline (speedup 1.0000x reference)
import functools

import jax
import jax.numpy as jnp
from jax import lax
from jax.experimental import pallas as pl
from jax.experimental.pallas import tpu as pltpu
from jax.experimental.pallas import tpu_sc as plsc

E = 8
TOP_K = 2
D_MODEL = 768
D_FF = 3072
SEQ = 2048
CAP = (SEQ * TOP_K // E) * 2
CAPB = CAP + 128
NROWS = E * CAPB
OVERFLOW = CAP

NC, NS = 2, 16
NW = NC * NS
TPW = SEQ // NW
FF_TILE = 768
N_FT = D_FF // FF_TILE


def _router_body(x_ref, gw_ref, slot_ref, w_ref, cnt_ref, pos_ref):
    x = x_ref[...]
    gw = gw_ref[...]
    logits = lax.dot_general(
        x, gw, (((1,), (1,)), ((), ())),
        preferred_element_type=jnp.float32,
        precision=lax.Precision.DEFAULT)
    m = jnp.max(logits, axis=1, keepdims=True)
    p = jnp.exp(logits - m)
    p = p / jnp.sum(p, axis=1, keepdims=True)

    e_ids = lax.broadcasted_iota(jnp.int32, (SEQ, E), 1)
    p0 = jnp.max(p, axis=1, keepdims=True)
    i0 = jnp.min(jnp.where(p == p0, e_ids, E), axis=1, keepdims=True)
    sel0 = e_ids == i0
    pm = jnp.where(sel0, -1.0, p)
    p1 = jnp.max(pm, axis=1, keepdims=True)
    i1 = jnp.min(jnp.where(pm == p1, e_ids, E), axis=1, keepdims=True)
    sel1 = e_ids == i1
    s = p0 + p1
    w0 = p0 / s
    w1 = p1 / s

    mask = jnp.where(sel0 | sel1, 1.0, 0.0)
    r = lax.broadcasted_iota(jnp.int32, (128, 128), 0)
    c = lax.broadcasted_iota(jnp.int32, (128, 128), 1)
    tril = jnp.where(r >= c, 1.0, 0.0)
    carry = jnp.zeros((1, E), jnp.float32)
    for b in range(SEQ // 128):
        chunk = mask[b * 128:(b + 1) * 128, :]
        inc = lax.dot_general(tril, chunk, (((1,), (0,)), ((), ())),
                              preferred_element_type=jnp.float32)
        pos_ref[b * 128:(b + 1) * 128, :] = inc + carry - 1.0
        carry = carry + jnp.sum(chunk, axis=0, keepdims=True)
    posm = pos_ref[...]

    pos0 = jnp.sum(jnp.where(sel0, posm, 0.0), axis=1, keepdims=True)
    pos1 = jnp.sum(jnp.where(sel1, posm, 0.0), axis=1, keepdims=True)
    pos0 = jnp.minimum(pos0, float(OVERFLOW)).astype(jnp.int32)
    pos1 = jnp.minimum(pos1, float(OVERFLOW)).astype(jnp.int32)
    slot0 = i0 * CAPB + pos0
    slot1 = i1 * CAPB + pos1
    slot_ref[0, :] = slot0[:, 0]
    slot_ref[1, :] = slot1[:, 0]
    w_ref[0, :] = w0[:, 0]
    w_ref[1, :] = w1[:, 0]
    cnt_ref[...] = jnp.minimum(carry, float(CAP)).astype(jnp.int32)


def _router(x, gate_w):
    return pl.pallas_call(
        _router_body,
        out_shape=(
            jax.ShapeDtypeStruct((TOP_K, SEQ), jnp.int32),
            jax.ShapeDtypeStruct((TOP_K, SEQ), jnp.float32),
            jax.ShapeDtypeStruct((1, E), jnp.int32),
        ),
        scratch_shapes=[pltpu.VMEM((SEQ, E), jnp.float32)],
        compiler_params=pltpu.CompilerParams(skip_device_barrier=True),
    )(x, gate_w)


def _dispatch_body(x_hbm, slot_hbm, w_hbm, xe_hbm, scale_hbm,
                   idx0_v, idx1_v, rows_v, slots_v, ws_v, scale_v, zrow_v, sem):
    wid = lax.axis_index("s") * NC + lax.axis_index("c")
    base = wid * TPW
    pltpu.sync_copy(slot_hbm.at[0, pl.ds(base, TPW)], idx0_v)
    pltpu.sync_copy(slot_hbm.at[1, pl.ds(base, TPW)], idx1_v)
    pltpu.sync_copy(x_hbm.at[pl.ds(base, TPW)], rows_v)
    pltpu.async_copy(rows_v, xe_hbm.at[idx0_v], sem).wait()
    pltpu.async_copy(rows_v, xe_hbm.at[idx1_v], sem).wait()

    @pl.when(wid == 0)
    def _():
        pltpu.sync_copy(slot_hbm, slots_v)
        pltpu.sync_copy(w_hbm, ws_v)

        def zero_it(i, _):
            scale_v[pl.ds(i * 16, 16)] = jnp.zeros((16,), jnp.float32)
            return 0
        lax.fori_loop(0, NROWS // 16, zero_it, 0)

        def scat(i, _):
            k = i // (SEQ // 16)
            j = i % (SEQ // 16)
            sv = slots_v[k, pl.ds(j * 16, 16)]
            wv = ws_v[k, pl.ds(j * 16, 16)]
            plsc.store_scatter(scale_v, [sv], wv)
            return 0
        lax.fori_loop(0, TOP_K * SEQ // 16, scat, 0)
        pltpu.sync_copy(scale_v, scale_hbm)

    def zrow(i, _):
        zrow_v[pl.ds(i * 16, 16)] = jnp.zeros((16,), jnp.float32)
        return 0
    lax.fori_loop(0, D_MODEL // 16, zrow, 0)
    plsc.subcore_barrier()

    @pl.when(wid < E)
    def _():
        pltpu.sync_copy(zrow_v, xe_hbm.at[wid * CAPB + OVERFLOW])


def _dispatch(x, slot, w):
    mesh = plsc.VectorSubcoreMesh(core_axis_name="c", subcore_axis_name="s",
                                  num_cores=NC, num_subcores=NS)
    return pl.kernel(
        _dispatch_body,
        out_type=(
            jax.ShapeDtypeStruct((NROWS, D_MODEL), jnp.float32),
            jax.ShapeDtypeStruct((NROWS,), jnp.float32),
        ),
        mesh=mesh,
        compiler_params=pltpu.CompilerParams(needs_layout_passes=False, skip_device_barrier=True),
        scratch_types=[
            pltpu.VMEM((TPW,), jnp.int32),
            pltpu.VMEM((TPW,), jnp.int32),
            pltpu.VMEM((TPW, D_MODEL), jnp.float32),
            pltpu.VMEM((TOP_K, SEQ), jnp.int32),
            pltpu.VMEM((TOP_K, SEQ), jnp.float32),
            pltpu.VMEM((NROWS,), jnp.float32),
            pltpu.VMEM((D_MODEL,), jnp.float32),
            pltpu.SemaphoreType.DMA,
        ],
    )(x, slot, w)


RB = CAPB // 2
N_RB = CAPB // RB


def _expert_body(cnt_ref, xe_ref, w1_ref, w3_ref, w2_ref, sc_ref, out_ref,
                 acc_ref):
    e = pl.program_id(0)
    f = pl.program_id(1)
    cnt = cnt_ref[0, e]
    w1b = w1_ref[0]
    w3b = w3_ref[0]
    w2b = w2_ref[0]

    for rb in range(N_RB):
        @pl.when(rb * RB <= cnt)
        def _():
            rs = pl.ds(rb * RB, RB)
            x = xe_ref[rs, :]
            h1 = lax.dot_general(x, w1b, (((1,), (1,)), ((), ())),
                                 preferred_element_type=jnp.float32,
                                 precision=lax.Precision.DEFAULT)
            h3 = lax.dot_general(x, w3b, (((1,), (1,)), ((), ())),
                                 preferred_element_type=jnp.float32,
                                 precision=lax.Precision.DEFAULT)
            h = (h1 * jax.nn.sigmoid(h1)) * h3
            y = lax.dot_general(h, w2b, (((1,), (1,)), ((), ())),
                                preferred_element_type=jnp.float32,
                                precision=lax.Precision.DEFAULT)

            @pl.when(f == 0)
            def _():
                acc_ref[rs, :] = y

            @pl.when(f > 0)
            def _():
                acc_ref[rs, :] += y

            @pl.when(f == N_FT - 1)
            def _():
                out_ref[rs, :] = acc_ref[rs, :] * sc_ref[rs, :]


def _experts(counts, xe, w1, w3, w2, scale2d):
    return pl.pallas_call(
        _expert_body,
        grid=(E, N_FT),
        in_specs=[
            pl.BlockSpec(memory_space=pltpu.SMEM),
            pl.BlockSpec((CAPB, D_MODEL), lambda e, f: (e, 0)),
            pl.BlockSpec((1, FF_TILE, D_MODEL), lambda e, f: (e, f, 0)),
            pl.BlockSpec((1, FF_TILE, D_MODEL), lambda e, f: (e, f, 0)),
            pl.BlockSpec((1, D_MODEL, FF_TILE), lambda e, f: (e, 0, f)),
            pl.BlockSpec((CAPB, 1), lambda e, f: (e, 0)),
        ],
        out_specs=pl.BlockSpec((CAPB, D_MODEL), lambda e, f: (e, 0)),
        out_shape=jax.ShapeDtypeStruct((NROWS, D_MODEL), jnp.float32),
        scratch_shapes=[pltpu.VMEM((CAPB, D_MODEL), jnp.float32)],
        compiler_params=pltpu.CompilerParams(
            dimension_semantics=("arbitrary", "arbitrary"),
            skip_device_barrier=True),
    )(counts, xe, w1, w3, w2, scale2d)


def _combine_body(ye_hbm, slot_hbm, out_hbm, idx0_v, idx1_v, r0_v, r1_v, sem):
    wid = lax.axis_index("s") * NC + lax.axis_index("c")
    half = TPW // 2
    for h in range(2):
        base = wid * TPW + h * half
        pltpu.sync_copy(slot_hbm.at[0, pl.ds(base, half)], idx0_v)
        pltpu.sync_copy(slot_hbm.at[1, pl.ds(base, half)], idx1_v)
        pltpu.async_copy(ye_hbm.at[idx0_v], r0_v, sem).wait()
        pltpu.async_copy(ye_hbm.at[idx1_v], r1_v, sem).wait()

        def addrow(t, _):
            for j in range(D_MODEL // 16):
                sl = pl.ds(j * 16, 16)
                r0_v[t, sl] += r1_v[t, sl]
            return 0
        lax.fori_loop(0, half, addrow, 0)
        pltpu.sync_copy(r0_v, out_hbm.at[pl.ds(base, half)])


def _combine(ye, slot):
    mesh = plsc.VectorSubcoreMesh(core_axis_name="c", subcore_axis_name="s",
                                  num_cores=NC, num_subcores=NS)
    half = TPW // 2
    return pl.kernel(
        _combine_body,
        out_type=jax.ShapeDtypeStruct((SEQ, D_MODEL), jnp.float32),
        mesh=mesh,
        scratch_types=[
            pltpu.VMEM((half,), jnp.int32),
            pltpu.VMEM((half,), jnp.int32),
            pltpu.VMEM((half, D_MODEL), jnp.float32),
            pltpu.VMEM((half, D_MODEL), jnp.float32),
            pltpu.SemaphoreType.DMA,
        ],
        compiler_params=pltpu.CompilerParams(skip_device_barrier=True),
    )(ye, slot)


def kernel(data, gate_w, w1, w2, w3):
    B, S, D = data.shape
    x = data.reshape(S, D)
    slot, w, counts = _router(x, gate_w)
    xe, scale = _dispatch(x, slot, w)
    ye = _experts(counts, xe, w1, w3, w2, scale.reshape(NROWS, 1))
    out = _combine(ye, slot)
    return out.reshape(B, S, D)


# speedup vs baseline: 1.0017x; 1.0017x over previous
import functools

import jax
import jax.numpy as jnp
from jax import lax
from jax.experimental import pallas as pl
from jax.experimental.pallas import tpu as pltpu
from jax.experimental.pallas import tpu_sc as plsc

E = 8
TOP_K = 2
D_MODEL = 768
D_FF = 3072
SEQ = 2048
CAP = (SEQ * TOP_K // E) * 2
CAPB = CAP + 128
NROWS = E * CAPB
OVERFLOW = CAP

NC, NS = 2, 16
NW = NC * NS
TPW = SEQ // NW
FF_TILE = 768
N_FT = D_FF // FF_TILE


def _router_body(x_ref, gw_ref, slot_ref, w_ref, cnt_ref, pos_ref):
    x = x_ref[...]
    gw = gw_ref[...]
    logits = lax.dot_general(
        x, gw, (((1,), (1,)), ((), ())),
        preferred_element_type=jnp.float32,
        precision=lax.Precision.DEFAULT)
    m = jnp.max(logits, axis=1, keepdims=True)
    p = jnp.exp(logits - m)
    p = p / jnp.sum(p, axis=1, keepdims=True)

    e_ids = lax.broadcasted_iota(jnp.int32, (SEQ, E), 1)
    p0 = jnp.max(p, axis=1, keepdims=True)
    i0 = jnp.min(jnp.where(p == p0, e_ids, E), axis=1, keepdims=True)
    sel0 = e_ids == i0
    pm = jnp.where(sel0, -1.0, p)
    p1 = jnp.max(pm, axis=1, keepdims=True)
    i1 = jnp.min(jnp.where(pm == p1, e_ids, E), axis=1, keepdims=True)
    sel1 = e_ids == i1
    s = p0 + p1
    w0 = p0 / s
    w1 = p1 / s

    mask = jnp.where(sel0 | sel1, 1.0, 0.0)
    r = lax.broadcasted_iota(jnp.int32, (128, 128), 0)
    c = lax.broadcasted_iota(jnp.int32, (128, 128), 1)
    tril = jnp.where(r >= c, 1.0, 0.0)
    carry = jnp.zeros((1, E), jnp.float32)
    for b in range(SEQ // 128):
        chunk = mask[b * 128:(b + 1) * 128, :]
        inc = lax.dot_general(tril, chunk, (((1,), (0,)), ((), ())),
                              preferred_element_type=jnp.float32)
        pos_ref[b * 128:(b + 1) * 128, :] = inc + carry - 1.0
        carry = carry + jnp.sum(chunk, axis=0, keepdims=True)
    posm = pos_ref[...]

    pos0 = jnp.sum(jnp.where(sel0, posm, 0.0), axis=1, keepdims=True)
    pos1 = jnp.sum(jnp.where(sel1, posm, 0.0), axis=1, keepdims=True)
    pos0 = jnp.minimum(pos0, float(OVERFLOW)).astype(jnp.int32)
    pos1 = jnp.minimum(pos1, float(OVERFLOW)).astype(jnp.int32)
    slot0 = i0 * CAPB + pos0
    slot1 = i1 * CAPB + pos1
    slot_ref[0, :] = slot0[:, 0]
    slot_ref[1, :] = slot1[:, 0]
    w_ref[0, :] = w0[:, 0]
    w_ref[1, :] = w1[:, 0]
    cnt_ref[...] = jnp.minimum(carry, float(CAP)).astype(jnp.int32)


def _router(x, gate_w):
    return pl.pallas_call(
        _router_body,
        out_shape=(
            jax.ShapeDtypeStruct((TOP_K, SEQ), jnp.int32),
            jax.ShapeDtypeStruct((TOP_K, SEQ), jnp.float32),
            jax.ShapeDtypeStruct((1, E), jnp.int32),
        ),
        scratch_shapes=[pltpu.VMEM((SEQ, E), jnp.float32)],
    )(x, gate_w)


def _dispatch_body(x_hbm, slot_hbm, w_hbm, xe_hbm, scale_hbm,
                   idx0_v, idx1_v, rows_v, slots_v, ws_v, scale_v, zrow_v, sem):
    wid = lax.axis_index("s") * NC + lax.axis_index("c")
    base = wid * TPW
    pltpu.sync_copy(slot_hbm.at[0, pl.ds(base, TPW)], idx0_v)
    pltpu.sync_copy(slot_hbm.at[1, pl.ds(base, TPW)], idx1_v)
    pltpu.sync_copy(x_hbm.at[pl.ds(base, TPW)], rows_v)
    pltpu.async_copy(rows_v, xe_hbm.at[idx0_v], sem).wait()
    pltpu.async_copy(rows_v, xe_hbm.at[idx1_v], sem).wait()

    @pl.when(wid == 0)
    def _():
        pltpu.sync_copy(slot_hbm, slots_v)
        pltpu.sync_copy(w_hbm, ws_v)

        def zero_it(i, _):
            scale_v[pl.ds(i * 16, 16)] = jnp.zeros((16,), jnp.float32)
            return 0
        lax.fori_loop(0, NROWS // 16, zero_it, 0)

        def scat(i, _):
            k = i // (SEQ // 16)
            j = i % (SEQ // 16)
            sv = slots_v[k, pl.ds(j * 16, 16)]
            wv = ws_v[k, pl.ds(j * 16, 16)]
            plsc.store_scatter(scale_v, [sv], wv)
            return 0
        lax.fori_loop(0, TOP_K * SEQ // 16, scat, 0)
        pltpu.sync_copy(scale_v, scale_hbm)

    def zrow(i, _):
        zrow_v[pl.ds(i * 16, 16)] = jnp.zeros((16,), jnp.float32)
        return 0
    lax.fori_loop(0, D_MODEL // 16, zrow, 0)
    plsc.subcore_barrier()

    @pl.when(wid < E)
    def _():
        pltpu.sync_copy(zrow_v, xe_hbm.at[wid * CAPB + OVERFLOW])


def _dispatch(x, slot, w):
    mesh = plsc.VectorSubcoreMesh(core_axis_name="c", subcore_axis_name="s",
                                  num_cores=NC, num_subcores=NS)
    return pl.kernel(
        _dispatch_body,
        out_type=(
            jax.ShapeDtypeStruct((NROWS, D_MODEL), jnp.float32),
            jax.ShapeDtypeStruct((NROWS,), jnp.float32),
        ),
        mesh=mesh,
        compiler_params=pltpu.CompilerParams(needs_layout_passes=False),
        scratch_types=[
            pltpu.VMEM((TPW,), jnp.int32),
            pltpu.VMEM((TPW,), jnp.int32),
            pltpu.VMEM((TPW, D_MODEL), jnp.float32),
            pltpu.VMEM((TOP_K, SEQ), jnp.int32),
            pltpu.VMEM((TOP_K, SEQ), jnp.float32),
            pltpu.VMEM((NROWS,), jnp.float32),
            pltpu.VMEM((D_MODEL,), jnp.float32),
            pltpu.SemaphoreType.DMA,
        ],
    )(x, slot, w)


RB = CAPB // 2
N_RB = CAPB // RB


def _expert_body(cnt_ref, xe_ref, w1_ref, w3_ref, w2_ref, sc_ref, out_ref,
                 acc_ref):
    e = pl.program_id(0)
    f = pl.program_id(1)
    cnt = cnt_ref[0, e]
    w1b = w1_ref[0]
    w3b = w3_ref[0]
    w2b = w2_ref[0]

    for rb in range(N_RB):
        @pl.when(rb * RB <= cnt)
        def _():
            rs = pl.ds(rb * RB, RB)
            x = xe_ref[rs, :]
            h1 = lax.dot_general(x, w1b, (((1,), (1,)), ((), ())),
                                 preferred_element_type=jnp.float32,
                                 precision=lax.Precision.DEFAULT)
            h3 = lax.dot_general(x, w3b, (((1,), (1,)), ((), ())),
                                 preferred_element_type=jnp.float32,
                                 precision=lax.Precision.DEFAULT)
            h = (h1 * jax.nn.sigmoid(h1)) * h3
            y = lax.dot_general(h, w2b, (((1,), (1,)), ((), ())),
                                preferred_element_type=jnp.float32,
                                precision=lax.Precision.DEFAULT)

            @pl.when(f == 0)
            def _():
                acc_ref[rs, :] = y

            @pl.when(f > 0)
            def _():
                acc_ref[rs, :] += y

            @pl.when(f == N_FT - 1)
            def _():
                out_ref[rs, :] = acc_ref[rs, :] * sc_ref[rs, :]


def _experts(counts, xe, w1, w3, w2, scale2d):
    return pl.pallas_call(
        _expert_body,
        grid=(E, N_FT),
        in_specs=[
            pl.BlockSpec(memory_space=pltpu.SMEM),
            pl.BlockSpec((CAPB, D_MODEL), lambda e, f: (e, 0)),
            pl.BlockSpec((1, FF_TILE, D_MODEL), lambda e, f: (e, f, 0)),
            pl.BlockSpec((1, FF_TILE, D_MODEL), lambda e, f: (e, f, 0)),
            pl.BlockSpec((1, D_MODEL, FF_TILE), lambda e, f: (e, 0, f)),
            pl.BlockSpec((CAPB, 1), lambda e, f: (e, 0)),
        ],
        out_specs=pl.BlockSpec((CAPB, D_MODEL), lambda e, f: (e, 0)),
        out_shape=jax.ShapeDtypeStruct((NROWS, D_MODEL), jnp.float32),
        scratch_shapes=[pltpu.VMEM((CAPB, D_MODEL), jnp.float32)],
        compiler_params=pltpu.CompilerParams(
            dimension_semantics=("arbitrary", "arbitrary")),
    )(counts, xe, w1, w3, w2, scale2d)


def _combine_body(ye_hbm, slot_hbm, out_hbm, idx0_v, idx1_v, r0_v, r1_v, sem):
    wid = lax.axis_index("s") * NC + lax.axis_index("c")
    half = TPW // 2
    for h in range(2):
        base = wid * TPW + h * half
        pltpu.sync_copy(slot_hbm.at[0, pl.ds(base, half)], idx0_v)
        pltpu.sync_copy(slot_hbm.at[1, pl.ds(base, half)], idx1_v)
        pltpu.async_copy(ye_hbm.at[idx0_v], r0_v, sem).wait()
        pltpu.async_copy(ye_hbm.at[idx1_v], r1_v, sem).wait()

        def addrow(t, _):
            for j in range(D_MODEL // 16):
                sl = pl.ds(j * 16, 16)
                r0_v[t, sl] += r1_v[t, sl]
            return 0
        lax.fori_loop(0, half, addrow, 0)
        pltpu.sync_copy(r0_v, out_hbm.at[pl.ds(base, half)])


def _combine(ye, slot):
    mesh = plsc.VectorSubcoreMesh(core_axis_name="c", subcore_axis_name="s",
                                  num_cores=NC, num_subcores=NS)
    half = TPW // 2
    return pl.kernel(
        _combine_body,
        out_type=jax.ShapeDtypeStruct((SEQ, D_MODEL), jnp.float32),
        mesh=mesh,
        scratch_types=[
            pltpu.VMEM((half,), jnp.int32),
            pltpu.VMEM((half,), jnp.int32),
            pltpu.VMEM((half, D_MODEL), jnp.float32),
            pltpu.VMEM((half, D_MODEL), jnp.float32),
            pltpu.SemaphoreType.DMA,
        ],
    )(ye, slot)


def kernel(data, gate_w, w1, w2, w3):
    B, S, D = data.shape
    x = data.reshape(S, D)
    slot, w, counts = _router(x, gate_w)
    xe, scale = _dispatch(x, slot, w)
    ye = _experts(counts, xe, w1, w3, w2, scale.reshape(NROWS, 1))
    out = _combine(ye, slot)
    return out.reshape(B, S, D)


# speedup vs baseline: 1.0271x; 1.0254x over previous
import functools

import jax
import jax.numpy as jnp
from jax import lax
from jax.experimental import pallas as pl
from jax.experimental.pallas import tpu as pltpu
from jax.experimental.pallas import tpu_sc as plsc

E = 8
TOP_K = 2
D_MODEL = 768
D_FF = 3072
SEQ = 2048
CAP = (SEQ * TOP_K // E) * 2
CAPB = CAP + 128
NROWS = E * CAPB
OVERFLOW = CAP

NC, NS = 2, 16
NW = NC * NS
TPW = SEQ // NW
FF_TILE = 768
N_FT = D_FF // FF_TILE


def _router_body(x_ref, gw_ref, slot_ref, w_ref, cnt_ref, pos_ref):
    x = x_ref[...]
    gw = gw_ref[...]
    logits = lax.dot_general(
        x, gw, (((1,), (1,)), ((), ())),
        preferred_element_type=jnp.float32,
        precision=lax.Precision.DEFAULT)
    m = jnp.max(logits, axis=1, keepdims=True)
    p = jnp.exp(logits - m)
    p = p / jnp.sum(p, axis=1, keepdims=True)

    e_ids = lax.broadcasted_iota(jnp.int32, (SEQ, E), 1)
    p0 = jnp.max(p, axis=1, keepdims=True)
    i0 = jnp.min(jnp.where(p == p0, e_ids, E), axis=1, keepdims=True)
    sel0 = e_ids == i0
    pm = jnp.where(sel0, -1.0, p)
    p1 = jnp.max(pm, axis=1, keepdims=True)
    i1 = jnp.min(jnp.where(pm == p1, e_ids, E), axis=1, keepdims=True)
    sel1 = e_ids == i1
    s = p0 + p1
    w0 = p0 / s
    w1 = p1 / s

    mask = jnp.where(sel0 | sel1, 1.0, 0.0)
    r = lax.broadcasted_iota(jnp.int32, (128, 128), 0)
    c = lax.broadcasted_iota(jnp.int32, (128, 128), 1)
    tril = jnp.where(r >= c, 1.0, 0.0)
    carry = jnp.zeros((1, E), jnp.float32)
    for b in range(SEQ // 128):
        chunk = mask[b * 128:(b + 1) * 128, :]
        inc = lax.dot_general(tril, chunk, (((1,), (0,)), ((), ())),
                              preferred_element_type=jnp.float32)
        pos_ref[b * 128:(b + 1) * 128, :] = inc + carry - 1.0
        carry = carry + jnp.sum(chunk, axis=0, keepdims=True)
    posm = pos_ref[...]

    pos0 = jnp.sum(jnp.where(sel0, posm, 0.0), axis=1, keepdims=True)
    pos1 = jnp.sum(jnp.where(sel1, posm, 0.0), axis=1, keepdims=True)
    pos0 = jnp.minimum(pos0, float(OVERFLOW)).astype(jnp.int32)
    pos1 = jnp.minimum(pos1, float(OVERFLOW)).astype(jnp.int32)
    slot0 = i0 * CAPB + pos0
    slot1 = i1 * CAPB + pos1
    slot_ref[0, :] = slot0[:, 0]
    slot_ref[1, :] = slot1[:, 0]
    w_ref[0, :] = w0[:, 0]
    w_ref[1, :] = w1[:, 0]
    cnt_ref[...] = jnp.minimum(carry, float(CAP)).astype(jnp.int32)


def _router(x, gate_w):
    return pl.pallas_call(
        _router_body,
        out_shape=(
            jax.ShapeDtypeStruct((TOP_K, SEQ), jnp.int32),
            jax.ShapeDtypeStruct((TOP_K, SEQ), jnp.float32),
            jax.ShapeDtypeStruct((1, E), jnp.int32),
        ),
        scratch_shapes=[pltpu.VMEM((SEQ, E), jnp.float32)],
    )(x, gate_w)


def _dispatch_body(x_hbm, slot_hbm, w_hbm, xe_hbm, scale_hbm,
                   idx0_v, idx1_v, rows_v, slots_v, ws_v, scale_v, zrow_v, sem):
    wid = lax.axis_index("s") * NC + lax.axis_index("c")
    base = wid * TPW
    pltpu.sync_copy(slot_hbm.at[0, pl.ds(base, TPW)], idx0_v)
    pltpu.sync_copy(slot_hbm.at[1, pl.ds(base, TPW)], idx1_v)
    pltpu.sync_copy(x_hbm.at[pl.ds(base, TPW)], rows_v)
    d0 = pltpu.async_copy(rows_v, xe_hbm.at[idx0_v], sem)
    d1 = pltpu.async_copy(rows_v, xe_hbm.at[idx1_v], sem)
    d0.wait()
    d1.wait()

    @pl.when(wid == 0)
    def _():
        pltpu.sync_copy(slot_hbm, slots_v)
        pltpu.sync_copy(w_hbm, ws_v)

        def scat(i, _):
            k = i // (SEQ // 16)
            j = i % (SEQ // 16)
            sv = slots_v[k, pl.ds(j * 16, 16)]
            wv = ws_v[k, pl.ds(j * 16, 16)]
            plsc.store_scatter(scale_v, [sv], wv)
            return 0
        lax.fori_loop(0, TOP_K * SEQ // 16, scat, 0)
        pltpu.sync_copy(scale_v, scale_hbm)

    def zrow(i, _):
        zrow_v[pl.ds(i * 16, 16)] = jnp.zeros((16,), jnp.float32)
        return 0
    lax.fori_loop(0, D_MODEL // 16, zrow, 0)
    plsc.subcore_barrier()

    @pl.when(wid < E)
    def _():
        pltpu.sync_copy(zrow_v, xe_hbm.at[wid * CAPB + OVERFLOW])


def _dispatch(x, slot, w):
    mesh = plsc.VectorSubcoreMesh(core_axis_name="c", subcore_axis_name="s",
                                  num_cores=NC, num_subcores=NS)
    return pl.kernel(
        _dispatch_body,
        out_type=(
            jax.ShapeDtypeStruct((NROWS, D_MODEL), jnp.float32),
            jax.ShapeDtypeStruct((NROWS,), jnp.float32),
        ),
        mesh=mesh,
        compiler_params=pltpu.CompilerParams(needs_layout_passes=False),
        scratch_types=[
            pltpu.VMEM((TPW,), jnp.int32),
            pltpu.VMEM((TPW,), jnp.int32),
            pltpu.VMEM((TPW, D_MODEL), jnp.float32),
            pltpu.VMEM((TOP_K, SEQ), jnp.int32),
            pltpu.VMEM((TOP_K, SEQ), jnp.float32),
            pltpu.VMEM((NROWS,), jnp.float32),
            pltpu.VMEM((D_MODEL,), jnp.float32),
            pltpu.SemaphoreType.DMA,
        ],
    )(x, slot, w)


RB = CAPB // 2
N_RB = CAPB // RB


def _expert_body(cnt_ref, xe_ref, w1_ref, w3_ref, w2_ref, sc_ref, out_ref,
                 acc_ref):
    e = pl.program_id(0)
    f = pl.program_id(1)
    cnt = cnt_ref[0, e]
    w1b = w1_ref[0]
    w3b = w3_ref[0]
    w2b = w2_ref[0]

    for rb in range(N_RB):
        @pl.when(rb * RB <= cnt)
        def _():
            rs = pl.ds(rb * RB, RB)
            x = xe_ref[rs, :]
            h1 = lax.dot_general(x, w1b, (((1,), (1,)), ((), ())),
                                 preferred_element_type=jnp.float32,
                                 precision=lax.Precision.DEFAULT)
            h3 = lax.dot_general(x, w3b, (((1,), (1,)), ((), ())),
                                 preferred_element_type=jnp.float32,
                                 precision=lax.Precision.DEFAULT)
            h = (h1 * jax.nn.sigmoid(h1)) * h3
            y = lax.dot_general(h, w2b, (((1,), (1,)), ((), ())),
                                preferred_element_type=jnp.float32,
                                precision=lax.Precision.DEFAULT)

            @pl.when(f == 0)
            def _():
                acc_ref[rs, :] = y

            @pl.when(f > 0)
            def _():
                acc_ref[rs, :] += y

            @pl.when(f == N_FT - 1)
            def _():
                out_ref[rs, :] = acc_ref[rs, :] * sc_ref[rs, :]


def _experts(counts, xe, w1, w3, w2, scale2d):
    return pl.pallas_call(
        _expert_body,
        grid=(E, N_FT),
        in_specs=[
            pl.BlockSpec(memory_space=pltpu.SMEM),
            pl.BlockSpec((CAPB, D_MODEL), lambda e, f: (e, 0)),
            pl.BlockSpec((1, FF_TILE, D_MODEL), lambda e, f: (e, f, 0)),
            pl.BlockSpec((1, FF_TILE, D_MODEL), lambda e, f: (e, f, 0)),
            pl.BlockSpec((1, D_MODEL, FF_TILE), lambda e, f: (e, 0, f)),
            pl.BlockSpec((CAPB, 1), lambda e, f: (e, 0)),
        ],
        out_specs=pl.BlockSpec((CAPB, D_MODEL), lambda e, f: (e, 0)),
        out_shape=jax.ShapeDtypeStruct((NROWS, D_MODEL), jnp.float32),
        scratch_shapes=[pltpu.VMEM((CAPB, D_MODEL), jnp.float32)],
        compiler_params=pltpu.CompilerParams(
            dimension_semantics=("arbitrary", "arbitrary")),
    )(counts, xe, w1, w3, w2, scale2d)


def _combine_body(ye_hbm, slot_hbm, out_hbm, idx0_v, idx1_v, r0_v, r1_v, sem):
    wid = lax.axis_index("s") * NC + lax.axis_index("c")
    half = TPW // 2
    for h in range(2):
        base = wid * TPW + h * half
        pltpu.sync_copy(slot_hbm.at[0, pl.ds(base, half)], idx0_v)
        pltpu.sync_copy(slot_hbm.at[1, pl.ds(base, half)], idx1_v)
        d0 = pltpu.async_copy(ye_hbm.at[idx0_v], r0_v, sem)
        d1 = pltpu.async_copy(ye_hbm.at[idx1_v], r1_v, sem)
        d0.wait()
        d1.wait()

        def addrow(t, _):
            for j in range(D_MODEL // 16):
                sl = pl.ds(j * 16, 16)
                r0_v[t, sl] += r1_v[t, sl]
            return 0
        lax.fori_loop(0, half, addrow, 0)
        pltpu.sync_copy(r0_v, out_hbm.at[pl.ds(base, half)])


def _combine(ye, slot):
    mesh = plsc.VectorSubcoreMesh(core_axis_name="c", subcore_axis_name="s",
                                  num_cores=NC, num_subcores=NS)
    half = TPW // 2
    return pl.kernel(
        _combine_body,
        out_type=jax.ShapeDtypeStruct((SEQ, D_MODEL), jnp.float32),
        mesh=mesh,
        scratch_types=[
            pltpu.VMEM((half,), jnp.int32),
            pltpu.VMEM((half,), jnp.int32),
            pltpu.VMEM((half, D_MODEL), jnp.float32),
            pltpu.VMEM((half, D_MODEL), jnp.float32),
            pltpu.SemaphoreType.DMA,
        ],
    )(ye, slot)


def kernel(data, gate_w, w1, w2, w3):
    B, S, D = data.shape
    x = data.reshape(S, D)
    slot, w, counts = _router(x, gate_w)
    xe, scale = _dispatch(x, slot, w)
    ye = _experts(counts, xe, w1, w3, w2, scale.reshape(NROWS, 1))
    out = _combine(ye, slot)
    return out.reshape(B, S, D)
